# Initial kernel scaffold; baseline (speedup 1.0000x reference)
import jax
import jax.numpy as jnp
from jax.experimental import pallas as pl


def kernel(time, emb_weight, lin_w, lin_b):
    raise NotImplementedError("write your pallas kernel here")


# speedup vs baseline: 4.5347x; 4.5347x over previous
import functools
import math

import jax
import jax.numpy as jnp
from jax import lax
from jax.experimental import pallas as pl
from jax.experimental.pallas import tpu as pltpu
from jax.experimental.pallas import tpu_sc as plsc

N_HID = 128
MAX_LEN = 240
BATCH = 4096
HIST = 200

NC = 2
NS = 16
NW = NC * NS

B_TOTAL = BATCH * HIST
B_PER_W = B_TOTAL // NW
CHUNK = 128
N_CHUNKS = B_PER_W // CHUNK


def _fuse_tc_kernel(emb_ref, w_ref, b_ref, out_ref):
    e = emb_ref[...]
    w = w_ref[...]
    acc = lax.dot_general(
        e, w, (((1,), (1,)), ((), ())), preferred_element_type=jnp.float32
    )
    out_ref[...] = acc + b_ref[...]


def _build_fused_table(emb_weight, lin_w, lin_b):
    return pl.pallas_call(
        _fuse_tc_kernel,
        out_shape=jax.ShapeDtypeStruct((MAX_LEN, N_HID), jnp.float32),
    )(emb_weight, lin_w, lin_b.reshape(1, N_HID))


def _gather_body(table_hbm, idx_hbm, out_hbm, idx_v, rows_v, sem):
    wid = lax.axis_index("s") * NC + lax.axis_index("c")
    w_base = pl.multiple_of(wid * B_PER_W, CHUNK)

    @pl.loop(0, N_CHUNKS)
    def _chunk(i):
        base = pl.multiple_of(w_base + i * CHUNK, CHUNK)
        pltpu.sync_copy(idx_hbm.at[pl.ds(base, CHUNK)], idx_v)
        pltpu.async_copy(table_hbm.at[idx_v], rows_v, sem).wait()
        pltpu.sync_copy(rows_v, out_hbm.at[pl.ds(base, CHUNK)])


@functools.partial(
    pl.kernel,
    out_type=jax.ShapeDtypeStruct((B_TOTAL, N_HID), jnp.float32),
    mesh=plsc.VectorSubcoreMesh(core_axis_name="c", subcore_axis_name="s"),
    scratch_types=[
        pltpu.VMEM((CHUNK,), jnp.int32),
        pltpu.VMEM((CHUNK, N_HID), jnp.float32),
        pltpu.SemaphoreType.DMA,
    ],
)
def _sc_gather(table_hbm, idx_hbm, out_hbm, idx_v, rows_v, sem):
    _gather_body(table_hbm, idx_hbm, out_hbm, idx_v, rows_v, sem)


def kernel(time, emb_weight, lin_w, lin_b):
    fused = _build_fused_table(emb_weight, lin_w, lin_b)
    idx = time.reshape(B_TOTAL)
    out = _sc_gather(fused, idx)
    return out.reshape(BATCH, HIST, N_HID)


# speedup vs baseline: 4.5347x; 1.0000x over previous
import functools
import math

import jax
import jax.numpy as jnp
from jax import lax
from jax.experimental import pallas as pl
from jax.experimental.pallas import tpu as pltpu
from jax.experimental.pallas import tpu_sc as plsc

N_HID = 128
MAX_LEN = 240
BATCH = 4096
HIST = 200

NC = 2
NS = 16
NW = NC * NS

B_TOTAL = BATCH * HIST
B_PER_W = B_TOTAL // NW
CHUNK = 128
N_CHUNKS = B_PER_W // CHUNK
NBUF = 4
N_GROUPS = N_CHUNKS // NBUF


def _fuse_tc_kernel(emb_ref, w_ref, b_ref, out_ref):
    e = emb_ref[...]
    w = w_ref[...]
    acc = lax.dot_general(
        e, w, (((1,), (1,)), ((), ())), preferred_element_type=jnp.float32
    )
    out_ref[...] = acc + b_ref[...]


def _build_fused_table(emb_weight, lin_w, lin_b):
    return pl.pallas_call(
        _fuse_tc_kernel,
        out_shape=jax.ShapeDtypeStruct((MAX_LEN, N_HID), jnp.float32),
    )(emb_weight, lin_w, lin_b.reshape(1, N_HID))


def _gather_body(table_hbm, idx_hbm, out_hbm, idx_v, rows, gsems, wsems):
    wid = lax.axis_index("s") * NC + lax.axis_index("c")
    w_base = pl.multiple_of(wid * B_PER_W, CHUNK)

    pltpu.sync_copy(idx_hbm.at[pl.ds(w_base, B_PER_W)], idx_v)

    def start_gather(chunk, b):
        off = pl.multiple_of(chunk * CHUNK, CHUNK)
        pltpu.async_copy(table_hbm.at[idx_v.at[pl.ds(off, CHUNK)]], rows[b], gsems[b])

    def start_write(chunk, b):
        off = pl.multiple_of(w_base + chunk * CHUNK, CHUNK)
        pltpu.async_copy(rows[b], out_hbm.at[pl.ds(off, CHUNK)], wsems[b])

    def drain_gather(b):
        pltpu.make_async_copy(
            table_hbm.at[idx_v.at[pl.ds(0, CHUNK)]], rows[b], gsems[b]
        ).wait()

    def drain_write(b):
        pltpu.make_async_copy(
            rows[b], out_hbm.at[pl.ds(w_base, CHUNK)], wsems[b]
        ).wait()

    for b in range(NBUF):
        start_gather(b, b)

    @pl.loop(0, N_GROUPS - 1)
    def _group(g):
        base_chunk = g * NBUF
        for b in range(NBUF):
            drain_gather(b)
            start_write(base_chunk + b, b)
        for b in range(NBUF):
            drain_write(b)
            start_gather(base_chunk + NBUF + b, b)

    last = (N_GROUPS - 1) * NBUF
    for b in range(NBUF):
        drain_gather(b)
        start_write(last + b, b)
    for b in range(NBUF):
        drain_write(b)


@functools.partial(
    pl.kernel,
    out_type=jax.ShapeDtypeStruct((B_TOTAL, N_HID), jnp.float32),
    mesh=plsc.VectorSubcoreMesh(core_axis_name="c", subcore_axis_name="s"),
    scratch_types=[
        pltpu.VMEM((B_PER_W,), jnp.int32),
        [pltpu.VMEM((CHUNK, N_HID), jnp.float32) for _ in range(NBUF)],
        [pltpu.SemaphoreType.DMA for _ in range(NBUF)],
        [pltpu.SemaphoreType.DMA for _ in range(NBUF)],
    ],
)
def _sc_gather(table_hbm, idx_hbm, out_hbm, idx_v, rows, gsems, wsems):
    _gather_body(table_hbm, idx_hbm, out_hbm, idx_v, rows, gsems, wsems)


def kernel(time, emb_weight, lin_w, lin_b):
    fused = _build_fused_table(emb_weight, lin_w, lin_b)
    idx = time.reshape(B_TOTAL)
    out = _sc_gather(fused, idx)
    return out.reshape(BATCH, HIST, N_HID)


# speedup vs baseline: 19.2448x; 4.2439x over previous
import functools
import math

import jax
import jax.numpy as jnp
from jax import lax
from jax.experimental import pallas as pl
from jax.experimental.pallas import tpu as pltpu
from jax.experimental.pallas import tpu_sc as plsc

N_HID = 128
MAX_LEN = 240
BATCH = 4096
HIST = 200

NC = 2
NS = 16
NW = NC * NS

B_TOTAL = BATCH * HIST
B_PER_W = B_TOTAL // NW
CHUNK = 128
N_CHUNKS = B_PER_W // CHUNK
NBUF = 5
N_GROUPS = N_CHUNKS // NBUF


def _fuse_tc_kernel(emb_ref, w_ref, b_ref, out_ref):
    e = emb_ref[...]
    w = w_ref[...]
    acc = lax.dot_general(
        e, w, (((1,), (1,)), ((), ())), preferred_element_type=jnp.float32
    )
    out_ref[...] = acc + b_ref[...]


def _build_fused_table(emb_weight, lin_w, lin_b):
    return pl.pallas_call(
        _fuse_tc_kernel,
        out_shape=jax.ShapeDtypeStruct((MAX_LEN, N_HID), jnp.float32),
    )(emb_weight, lin_w, lin_b.reshape(1, N_HID))


def _gather_body(table_hbm, idx_hbm, out_hbm, table_v, idx_v, rows, gsems, wsems):
    wid = lax.axis_index("s") * NC + lax.axis_index("c")
    w_base = pl.multiple_of(wid * B_PER_W, CHUNK)

    @pl.when(lax.axis_index("s") == 0)
    def _stage():
        pltpu.sync_copy(table_hbm, table_v)

    plsc.subcore_barrier()
    pltpu.sync_copy(idx_hbm.at[pl.ds(w_base, B_PER_W)], idx_v)

    def start_gather(chunk, b):
        off = pl.multiple_of(chunk * CHUNK, CHUNK)
        pltpu.async_copy(table_v.at[idx_v.at[pl.ds(off, CHUNK)]], rows[b], gsems[b])

    def start_write(chunk, b):
        off = pl.multiple_of(w_base + chunk * CHUNK, CHUNK)
        pltpu.async_copy(rows[b], out_hbm.at[pl.ds(off, CHUNK)], wsems[b])

    def drain_gather(b):
        pltpu.make_async_copy(
            table_v.at[idx_v.at[pl.ds(0, CHUNK)]], rows[b], gsems[b]
        ).wait()

    def drain_write(b):
        pltpu.make_async_copy(
            rows[b], out_hbm.at[pl.ds(w_base, CHUNK)], wsems[b]
        ).wait()

    for b in range(NBUF):
        start_gather(b, b)

    @pl.loop(0, N_GROUPS - 1)
    def _group(g):
        base_chunk = g * NBUF
        for b in range(NBUF):
            drain_gather(b)
            start_write(base_chunk + b, b)
        for b in range(NBUF):
            drain_write(b)
            start_gather(base_chunk + NBUF + b, b)

    last = (N_GROUPS - 1) * NBUF
    for b in range(NBUF):
        drain_gather(b)
        start_write(last + b, b)
    for b in range(NBUF):
        drain_write(b)


@functools.partial(
    pl.kernel,
    out_type=jax.ShapeDtypeStruct((B_TOTAL, N_HID), jnp.float32),
    mesh=plsc.VectorSubcoreMesh(core_axis_name="c", subcore_axis_name="s"),
    scratch_types=[
        pltpu.VMEM_SHARED((MAX_LEN, N_HID), jnp.float32),
        pltpu.VMEM((B_PER_W,), jnp.int32),
        [pltpu.VMEM((CHUNK, N_HID), jnp.float32) for _ in range(NBUF)],
        [pltpu.SemaphoreType.DMA for _ in range(NBUF)],
        [pltpu.SemaphoreType.DMA for _ in range(NBUF)],
    ],
)
def _sc_gather(table_hbm, idx_hbm, out_hbm, table_v, idx_v, rows, gsems, wsems):
    _gather_body(table_hbm, idx_hbm, out_hbm, table_v, idx_v, rows, gsems, wsems)


def kernel(time, emb_weight, lin_w, lin_b):
    fused = _build_fused_table(emb_weight, lin_w, lin_b)
    idx = time.reshape(B_TOTAL)
    out = _sc_gather(fused, idx)
    return out.reshape(BATCH, HIST, N_HID)


# speedup vs baseline: 19.3489x; 1.0054x over previous
import functools
import math

import jax
import jax.numpy as jnp
from jax import lax
from jax.experimental import pallas as pl
from jax.experimental.pallas import tpu as pltpu
from jax.experimental.pallas import tpu_sc as plsc

N_HID = 128
MAX_LEN = 240
BATCH = 4096
HIST = 200

NC = 2
NS = 16
NW = NC * NS

B_TOTAL = BATCH * HIST
B_PER_W = B_TOTAL // NW
CHUNK = 128
N_CHUNKS = B_PER_W // CHUNK
NBUF = 4
N_GROUPS = N_CHUNKS // NBUF


def _fuse_tc_kernel(emb_ref, w_ref, b_ref, out_ref):
    e = emb_ref[...]
    w = w_ref[...]
    acc = lax.dot_general(
        e, w, (((1,), (1,)), ((), ())), preferred_element_type=jnp.float32
    )
    out_ref[...] = acc + b_ref[...]


def _build_fused_table(emb_weight, lin_w, lin_b):
    return pl.pallas_call(
        _fuse_tc_kernel,
        out_shape=jax.ShapeDtypeStruct((MAX_LEN, N_HID), jnp.float32),
    )(emb_weight, lin_w, lin_b.reshape(1, N_HID))


def _gather_body(table_hbm, idx_hbm, out_hbm, table_v, idx_v, rows, gsems, wsems):
    wid = lax.axis_index("s") * NC + lax.axis_index("c")
    w_base = pl.multiple_of(wid * B_PER_W, CHUNK)

    @pl.when(lax.axis_index("s") == 0)
    def _stage():
        pltpu.sync_copy(table_hbm, table_v)

    plsc.subcore_barrier()
    pltpu.sync_copy(idx_hbm.at[pl.ds(w_base, B_PER_W)], idx_v)

    def start_gather(chunk, b):
        off = pl.multiple_of(chunk * CHUNK, CHUNK)
        pltpu.async_copy(table_v.at[idx_v.at[pl.ds(off, CHUNK)]], rows[b], gsems[b], priority=1)

    def start_write(chunk, b):
        off = pl.multiple_of(w_base + chunk * CHUNK, CHUNK)
        pltpu.async_copy(rows[b], out_hbm.at[pl.ds(off, CHUNK)], wsems[b])

    def drain_gather(b):
        pltpu.make_async_copy(
            table_v.at[idx_v.at[pl.ds(0, CHUNK)]], rows[b], gsems[b]
        ).wait()

    def drain_write(b):
        pltpu.make_async_copy(
            rows[b], out_hbm.at[pl.ds(w_base, CHUNK)], wsems[b]
        ).wait()

    for b in range(NBUF):
        start_gather(b, b)

    @pl.loop(0, N_GROUPS - 1)
    def _group(g):
        base_chunk = g * NBUF
        for b in range(NBUF):
            drain_gather(b)
            start_write(base_chunk + b, b)
        for b in range(NBUF):
            drain_write(b)
            start_gather(base_chunk + NBUF + b, b)

    last = (N_GROUPS - 1) * NBUF
    for b in range(NBUF):
        drain_gather(b)
        start_write(last + b, b)
    for b in range(NBUF):
        drain_write(b)


@functools.partial(
    pl.kernel,
    out_type=jax.ShapeDtypeStruct((B_TOTAL, N_HID), jnp.float32),
    mesh=plsc.VectorSubcoreMesh(core_axis_name="c", subcore_axis_name="s"),
    scratch_types=[
        pltpu.VMEM_SHARED((MAX_LEN, N_HID), jnp.float32),
        pltpu.VMEM((B_PER_W,), jnp.int32),
        [pltpu.VMEM((CHUNK, N_HID), jnp.float32) for _ in range(NBUF)],
        [pltpu.SemaphoreType.DMA for _ in range(NBUF)],
        [pltpu.SemaphoreType.DMA for _ in range(NBUF)],
    ],
)
def _sc_gather(table_hbm, idx_hbm, out_hbm, table_v, idx_v, rows, gsems, wsems):
    _gather_body(table_hbm, idx_hbm, out_hbm, table_v, idx_v, rows, gsems, wsems)


def kernel(time, emb_weight, lin_w, lin_b):
    fused = _build_fused_table(emb_weight, lin_w, lin_b)
    idx = time.reshape(B_TOTAL)
    out = _sc_gather(fused, idx)
    return out.reshape(BATCH, HIST, N_HID)
